# Initial kernel scaffold; baseline (speedup 1.0000x reference)
import jax
import jax.numpy as jnp
from jax.experimental import pallas as pl


def kernel(feat, edge_index, W1, b1, W2, b2):
    raise NotImplementedError("write your pallas kernel here")


# speedup vs baseline: 4.1602x; 4.1602x over previous
import functools

import jax
import jax.numpy as jnp
from jax import lax
from jax.experimental import pallas as pl
from jax.experimental.pallas import tpu as pltpu
from jax.experimental.pallas import tpu_sc as plsc

N = 10000
E = 160000
D = 256
DH = 128

NC = 2
NS = 16
CHUNK = 128
NCHUNK = E // CHUNK
CPT = -(-NCHUNK // NS)

_MESH = plsc.VectorSubcoreMesh(core_axis_name="c", subcore_axis_name="s")

_ZROWS = 2000


def _fill_zeros_2d(zbuf):
    @pl.loop(0, 16 * DH // 16)
    def _(i):
        zbuf[i // (DH // 16), pl.ds((i % (DH // 16)) * 16, 16)] = jnp.zeros(
            (16,), jnp.float32)


@functools.partial(
    pl.kernel,
    out_type=(jax.ShapeDtypeStruct((N,), jnp.float32),
              jax.ShapeDtypeStruct((N,), jnp.float32)),
    mesh=_MESH,
    scratch_types=[
        pltpu.VMEM_SHARED((N,), jnp.float32),
        pltpu.VMEM((CHUNK,), jnp.int32),
        pltpu.VMEM((CHUNK,), jnp.float32),
        pltpu.VMEM((_ZROWS,), jnp.float32),
    ],
)
def _deg_kernel(src_hbm, dst_hbm, do_hbm, di_hbm, acc, idx_v, ones_v, zline_v):
    c = lax.axis_index("c")
    s = lax.axis_index("s")

    @pl.loop(0, CHUNK // 16)
    def _(i):
        ones_v[pl.ds(i * 16, 16)] = jnp.ones((16,), jnp.float32)

    @pl.when(s < N // _ZROWS)
    def _():
        @pl.loop(0, _ZROWS // 16)
        def _(i):
            zline_v[pl.ds(i * 16, 16)] = jnp.zeros((16,), jnp.float32)
        pltpu.sync_copy(zline_v, acc.at[pl.ds(s * _ZROWS, _ZROWS)])

    plsc.subcore_barrier()

    @pl.loop(0, CPT)
    def _(j):
        ch = s + j * NS

        @pl.when(ch < NCHUNK)
        def _():
            @pl.when(c == 0)
            def _():
                pltpu.sync_copy(src_hbm.at[pl.ds(ch * CHUNK, CHUNK)], idx_v)

            @pl.when(c == 1)
            def _():
                pltpu.sync_copy(dst_hbm.at[pl.ds(ch * CHUNK, CHUNK)], idx_v)

            pltpu.sync_copy(ones_v, acc.at[idx_v], add=True)

    plsc.subcore_barrier()

    @pl.when(s < N // _ZROWS)
    def _():
        pltpu.sync_copy(acc.at[pl.ds(s * _ZROWS, _ZROWS)], zline_v)

        @pl.when(c == 0)
        def _():
            pltpu.sync_copy(zline_v, do_hbm.at[pl.ds(s * _ZROWS, _ZROWS)])

        @pl.when(c == 1)
        def _():
            pltpu.sync_copy(zline_v, di_hbm.at[pl.ds(s * _ZROWS, _ZROWS)])


_RPT = N // NS


@functools.partial(
    pl.kernel,
    out_type=(jax.ShapeDtypeStruct((N, DH), jnp.float32),
              jax.ShapeDtypeStruct((N, DH), jnp.float32)),
    mesh=_MESH,
    scratch_types=[
        pltpu.VMEM_SHARED((N, DH), jnp.float32),
        pltpu.VMEM((CHUNK,), jnp.int32),
        pltpu.VMEM((1, CHUNK), jnp.int32),
        pltpu.VMEM((CHUNK, DH), jnp.float32),
        pltpu.VMEM((16, DH), jnp.float32),
    ],
)
def _agg_kernel(h2_hbm, src_hbm, dst_hbm, out0_hbm, out1_hbm,
                acc, sidx_v, didx_v, gbuf_v, zbuf_v):
    c = lax.axis_index("c")
    s = lax.axis_index("s")

    @pl.when(s < N // _ZROWS)
    def _():
        _fill_zeros_2d(zbuf_v)

        @pl.loop(0, _ZROWS // 16)
        def _(j):
            pltpu.sync_copy(zbuf_v, acc.at[pl.ds(s * _ZROWS + j * 16, 16)])

    plsc.subcore_barrier()

    @pl.loop(0, CPT)
    def _(j):
        ch = s + j * NS

        @pl.when(ch < NCHUNK)
        def _():
            off = ch * CHUNK
            pltpu.sync_copy(src_hbm.at[pl.ds(off, CHUNK)], sidx_v)
            pltpu.sync_copy(dst_hbm.at[pl.ds(off, CHUNK)], didx_v.at[0])

            @pl.loop(0, CHUNK // 16)
            def _(i):
                sidx_v[pl.ds(i * 16, 16)] = sidx_v[pl.ds(i * 16, 16)] * 2 + c

            pltpu.sync_copy(h2_hbm.at[sidx_v], gbuf_v)
            pltpu.sync_copy(gbuf_v, acc.at[didx_v.at[0]], add=True)

    plsc.subcore_barrier()

    @pl.loop(0, -(-(N // 80) // NS))
    def _(j):
        blk = s + j * NS

        @pl.when(blk < N // 80)
        def _():
            base = blk * 80
            pltpu.sync_copy(acc.at[pl.ds(base, 80)], gbuf_v.at[pl.ds(0, 80)])

            @pl.when(c == 0)
            def _():
                pltpu.sync_copy(gbuf_v.at[pl.ds(0, 80)],
                                out0_hbm.at[pl.ds(base, 80)])

            @pl.when(c == 1)
            def _():
                pltpu.sync_copy(gbuf_v.at[pl.ds(0, 80)],
                                out1_hbm.at[pl.ds(base, 80)])


_BLK = 1000
_GRID = N // _BLK


def _norm(deg):
    return jnp.where(deg > 0, lax.rsqrt(jnp.maximum(deg, 1e-12)), 0.0)


def _mm1_body(x_ref, do_ref, w_ref, o_ref):
    ns = _norm(do_ref[...])
    o_ref[...] = jnp.dot(x_ref[...] * ns, w_ref[...],
                         preferred_element_type=jnp.float32)


_mm1 = pl.pallas_call(
    _mm1_body,
    grid=(_GRID,),
    in_specs=[
        pl.BlockSpec((_BLK, D), lambda i: (i, 0)),
        pl.BlockSpec((_BLK, 1), lambda i: (i, 0)),
        pl.BlockSpec((D, D), lambda i: (0, 0)),
    ],
    out_specs=pl.BlockSpec((_BLK, D), lambda i: (i, 0)),
    out_shape=jax.ShapeDtypeStruct((N, D), jnp.float32),
)


def _mid_body(a0_ref, a1_ref, di_ref, do_ref, b_ref, w_ref, o_ref):
    nd = _norm(di_ref[...])
    ns = _norm(do_ref[...])
    t0 = jnp.maximum(a0_ref[...] * nd + b_ref[0, :DH], 0.0) * ns
    t1 = jnp.maximum(a1_ref[...] * nd + b_ref[0, DH:], 0.0) * ns
    o_ref[...] = (
        jnp.dot(t0, w_ref[:DH, :], preferred_element_type=jnp.float32)
        + jnp.dot(t1, w_ref[DH:, :], preferred_element_type=jnp.float32))


_mid = pl.pallas_call(
    _mid_body,
    grid=(_GRID,),
    in_specs=[
        pl.BlockSpec((_BLK, DH), lambda i: (i, 0)),
        pl.BlockSpec((_BLK, DH), lambda i: (i, 0)),
        pl.BlockSpec((_BLK, 1), lambda i: (i, 0)),
        pl.BlockSpec((_BLK, 1), lambda i: (i, 0)),
        pl.BlockSpec((1, D), lambda i: (0, 0)),
        pl.BlockSpec((D, D), lambda i: (0, 0)),
    ],
    out_specs=pl.BlockSpec((_BLK, D), lambda i: (i, 0)),
    out_shape=jax.ShapeDtypeStruct((N, D), jnp.float32),
)


def _fin_body(a0_ref, a1_ref, di_ref, b_ref, o_ref):
    nd = _norm(di_ref[...])
    t0 = jnp.maximum(a0_ref[...] * nd + b_ref[0, :DH], 0.0)
    t1 = jnp.maximum(a1_ref[...] * nd + b_ref[0, DH:], 0.0)
    o_ref[...] = jnp.concatenate([t0, t1], axis=1)


_fin = pl.pallas_call(
    _fin_body,
    grid=(_GRID,),
    in_specs=[
        pl.BlockSpec((_BLK, DH), lambda i: (i, 0)),
        pl.BlockSpec((_BLK, DH), lambda i: (i, 0)),
        pl.BlockSpec((_BLK, 1), lambda i: (i, 0)),
        pl.BlockSpec((1, D), lambda i: (0, 0)),
    ],
    out_specs=pl.BlockSpec((_BLK, D), lambda i: (i, 0)),
    out_shape=jax.ShapeDtypeStruct((N, D), jnp.float32),
)


def kernel(feat, edge_index, W1, b1, W2, b2):
    ei = edge_index.astype(jnp.int32)
    src, dst = ei[0], ei[1]
    deg_out, deg_in = _deg_kernel(src, dst)
    do = deg_out.reshape(N, 1)
    di = deg_in.reshape(N, 1)
    h = _mm1(feat, do, W1)
    a0, a1 = _agg_kernel(h.reshape(2 * N, DH), src, dst)
    h2 = _mid(a0, a1, di, do, b1.reshape(1, D), W2)
    a0, a1 = _agg_kernel(h2.reshape(2 * N, DH), src, dst)
    return _fin(a0, a1, di, b2.reshape(1, D))


# speedup vs baseline: 8.5561x; 2.0567x over previous
import functools

import jax
import jax.numpy as jnp
from jax import lax
from jax.experimental import pallas as pl
from jax.experimental.pallas import tpu as pltpu
from jax.experimental.pallas import tpu_sc as plsc

N = 10000
E = 160000
D = 256
DH = 128

NC = 2
NS = 16
EPT = E // NS
GSZ = 80
NG = EPT // GSZ

_MESH = plsc.VectorSubcoreMesh(core_axis_name="c", subcore_axis_name="s")


_ZROWS = 2000


@functools.partial(
    pl.kernel,
    out_type=(jax.ShapeDtypeStruct((N,), jnp.float32),
              jax.ShapeDtypeStruct((N,), jnp.float32)),
    mesh=_MESH,
    scratch_types=[
        pltpu.VMEM_SHARED((N,), jnp.float32),
        pltpu.VMEM((EPT,), jnp.int32),
        pltpu.VMEM((EPT,), jnp.float32),
        pltpu.VMEM((_ZROWS,), jnp.float32),
    ],
)
def _deg_kernel(src_hbm, dst_hbm, do_hbm, di_hbm, acc, idx_v, ones_v, zline_v):
    c = lax.axis_index("c")
    s = lax.axis_index("s")

    @pl.loop(0, EPT // 16)
    def _(i):
        ones_v[pl.ds(i * 16, 16)] = jnp.ones((16,), jnp.float32)

    @pl.when(s < N // _ZROWS)
    def _():
        @pl.loop(0, _ZROWS // 16)
        def _(i):
            zline_v[pl.ds(i * 16, 16)] = jnp.zeros((16,), jnp.float32)
        pltpu.sync_copy(zline_v, acc.at[pl.ds(s * _ZROWS, _ZROWS)])

    plsc.subcore_barrier()

    @pl.when(c == 0)
    def _():
        pltpu.sync_copy(src_hbm.at[pl.ds(s * EPT, EPT)], idx_v)

    @pl.when(c == 1)
    def _():
        pltpu.sync_copy(dst_hbm.at[pl.ds(s * EPT, EPT)], idx_v)

    pltpu.sync_copy(ones_v, acc.at[idx_v], add=True)

    plsc.subcore_barrier()

    @pl.when(s < N // _ZROWS)
    def _():
        pltpu.sync_copy(acc.at[pl.ds(s * _ZROWS, _ZROWS)], zline_v)

        @pl.when(c == 0)
        def _():
            pltpu.sync_copy(zline_v, do_hbm.at[pl.ds(s * _ZROWS, _ZROWS)])

        @pl.when(c == 1)
        def _():
            pltpu.sync_copy(zline_v, di_hbm.at[pl.ds(s * _ZROWS, _ZROWS)])


_ORPB = 40
_ONB = N // _ORPB


@functools.partial(
    pl.kernel,
    out_type=(jax.ShapeDtypeStruct((N, DH), jnp.float32),
              jax.ShapeDtypeStruct((N, DH), jnp.float32)),
    mesh=_MESH,
    scratch_types=[
        pltpu.VMEM_SHARED((N, DH), jnp.float32),
        pltpu.VMEM((EPT,), jnp.int32),
        pltpu.VMEM((EPT,), jnp.int32),
        pltpu.VMEM((2, GSZ, DH), jnp.float32),
        pltpu.VMEM((_ORPB, DH), jnp.float32),
        pltpu.SemaphoreType.DMA,
        pltpu.SemaphoreType.DMA,
    ],
)
def _agg_kernel(h0_hbm, h1_hbm, src_hbm, dst_hbm, out0_hbm, out1_hbm,
                acc, sidx_v, didx_v, gbuf_v, zbuf_v, sem0, sem1):
    c = lax.axis_index("c")
    s = lax.axis_index("s")
    sems = (sem0, sem1)

    @pl.loop(0, _ORPB * (DH // 16))
    def _(i):
        zbuf_v[i // (DH // 16), pl.ds((i % (DH // 16)) * 16, 16)] = \
            jnp.zeros((16,), jnp.float32)

    @pl.loop(0, -(-_ONB // NS))
    def _(j):
        blk = s + j * NS

        @pl.when(blk < _ONB)
        def _():
            pltpu.sync_copy(zbuf_v, acc.at[pl.ds(blk * _ORPB, _ORPB)])

    pltpu.sync_copy(src_hbm.at[pl.ds(s * EPT, EPT)], sidx_v)
    pltpu.sync_copy(dst_hbm.at[pl.ds(s * EPT, EPT)], didx_v)

    plsc.subcore_barrier()

    def _gidx(g):
        return sidx_v.at[pl.ds(g * GSZ, GSZ)]

    def _gather(g, b):
        @pl.when(c == 0)
        def _():
            pltpu.make_async_copy(
                h0_hbm.at[_gidx(g)], gbuf_v.at[b], sems[b]).start()

        @pl.when(c == 1)
        def _():
            pltpu.make_async_copy(
                h1_hbm.at[_gidx(g)], gbuf_v.at[b], sems[b]).start()

    def _gwait(g, b):
        @pl.when(c == 0)
        def _():
            pltpu.make_async_copy(
                h0_hbm.at[_gidx(g)], gbuf_v.at[b], sems[b]).wait()

        @pl.when(c == 1)
        def _():
            pltpu.make_async_copy(
                h1_hbm.at[_gidx(g)], gbuf_v.at[b], sems[b]).wait()

    _gather(0, 0)

    @pl.loop(0, (NG + 1) // 2)
    def _(t):
        for b in range(2):
            g = t * 2 + b

            @pl.when(g + 1 < NG)
            def _():
                _gather(g + 1, 1 - b)

            @pl.when(g < NG)
            def _():
                _gwait(g, b)
                pltpu.sync_copy(
                    gbuf_v.at[b],
                    acc.at[didx_v.at[pl.ds(g * GSZ, GSZ)]], add=True)

    plsc.subcore_barrier()

    @pl.loop(0, -(-_ONB // NS))
    def _(j):
        blk = s + j * NS

        @pl.when(blk < _ONB)
        def _():
            base = blk * _ORPB
            pltpu.sync_copy(acc.at[pl.ds(base, _ORPB)], zbuf_v)

            @pl.when(c == 0)
            def _():
                pltpu.sync_copy(zbuf_v, out0_hbm.at[pl.ds(base, _ORPB)])

            @pl.when(c == 1)
            def _():
                pltpu.sync_copy(zbuf_v, out1_hbm.at[pl.ds(base, _ORPB)])


_BLK = 1000
_GRID = N // _BLK


def _norm(deg):
    return jnp.where(deg > 0, lax.rsqrt(jnp.maximum(deg, 1e-12)), 0.0)


def _mm1_body(x_ref, do_ref, w_ref, o0_ref, o1_ref):
    ns = _norm(do_ref[...])
    h = jnp.dot(x_ref[...] * ns, w_ref[...],
                preferred_element_type=jnp.float32)
    o0_ref[...] = h[:, :DH]
    o1_ref[...] = h[:, DH:]


_mm1 = pl.pallas_call(
    _mm1_body,
    grid=(_GRID,),
    in_specs=[
        pl.BlockSpec((_BLK, D), lambda i: (i, 0)),
        pl.BlockSpec((_BLK, 1), lambda i: (i, 0)),
        pl.BlockSpec((D, D), lambda i: (0, 0)),
    ],
    out_specs=[pl.BlockSpec((_BLK, DH), lambda i: (i, 0)),
               pl.BlockSpec((_BLK, DH), lambda i: (i, 0))],
    out_shape=(jax.ShapeDtypeStruct((N, DH), jnp.float32),
               jax.ShapeDtypeStruct((N, DH), jnp.float32)),
)


def _mid_body(a0_ref, a1_ref, di_ref, do_ref, b_ref, w_ref, o0_ref, o1_ref):
    nd = _norm(di_ref[...])
    ns = _norm(do_ref[...])
    t0 = jnp.maximum(a0_ref[...] * nd + b_ref[0, :DH], 0.0) * ns
    t1 = jnp.maximum(a1_ref[...] * nd + b_ref[0, DH:], 0.0) * ns
    h = (jnp.dot(t0, w_ref[:DH, :], preferred_element_type=jnp.float32)
         + jnp.dot(t1, w_ref[DH:, :], preferred_element_type=jnp.float32))
    o0_ref[...] = h[:, :DH]
    o1_ref[...] = h[:, DH:]


_mid = pl.pallas_call(
    _mid_body,
    grid=(_GRID,),
    in_specs=[
        pl.BlockSpec((_BLK, DH), lambda i: (i, 0)),
        pl.BlockSpec((_BLK, DH), lambda i: (i, 0)),
        pl.BlockSpec((_BLK, 1), lambda i: (i, 0)),
        pl.BlockSpec((_BLK, 1), lambda i: (i, 0)),
        pl.BlockSpec((1, D), lambda i: (0, 0)),
        pl.BlockSpec((D, D), lambda i: (0, 0)),
    ],
    out_specs=[pl.BlockSpec((_BLK, DH), lambda i: (i, 0)),
               pl.BlockSpec((_BLK, DH), lambda i: (i, 0))],
    out_shape=(jax.ShapeDtypeStruct((N, DH), jnp.float32),
               jax.ShapeDtypeStruct((N, DH), jnp.float32)),
)


def _fin_body(a0_ref, a1_ref, di_ref, b_ref, o_ref):
    nd = _norm(di_ref[...])
    t0 = jnp.maximum(a0_ref[...] * nd + b_ref[0, :DH], 0.0)
    t1 = jnp.maximum(a1_ref[...] * nd + b_ref[0, DH:], 0.0)
    o_ref[...] = jnp.concatenate([t0, t1], axis=1)


_fin = pl.pallas_call(
    _fin_body,
    grid=(_GRID,),
    in_specs=[
        pl.BlockSpec((_BLK, DH), lambda i: (i, 0)),
        pl.BlockSpec((_BLK, DH), lambda i: (i, 0)),
        pl.BlockSpec((_BLK, 1), lambda i: (i, 0)),
        pl.BlockSpec((1, D), lambda i: (0, 0)),
    ],
    out_specs=pl.BlockSpec((_BLK, D), lambda i: (i, 0)),
    out_shape=jax.ShapeDtypeStruct((N, D), jnp.float32),
)


def kernel(feat, edge_index, W1, b1, W2, b2):
    ei = edge_index.astype(jnp.int32)
    src, dst = ei[0], ei[1]
    deg_out, deg_in = _deg_kernel(src, dst)
    do = deg_out.reshape(N, 1)
    di = deg_in.reshape(N, 1)
    h0, h1 = _mm1(feat, do, W1)
    a0, a1 = _agg_kernel(h0, h1, src, dst)
    h0, h1 = _mid(a0, a1, di, do, b1.reshape(1, D), W2)
    a0, a1 = _agg_kernel(h0, h1, src, dst)
    return _fin(a0, a1, di, b2.reshape(1, D))


# speedup vs baseline: 9.2332x; 1.0791x over previous
import functools

import jax
import jax.numpy as jnp
from jax import lax
from jax.experimental import pallas as pl
from jax.experimental.pallas import tpu as pltpu
from jax.experimental.pallas import tpu_sc as plsc

N = 10000
E = 160000
D = 256
DH = 128

NC = 2
NS = 16
EPT = E // NS
GSZ = 40
NG = EPT // GSZ
NBUF = 4

_MESH = plsc.VectorSubcoreMesh(core_axis_name="c", subcore_axis_name="s")


_ZROWS = 2000


@functools.partial(
    pl.kernel,
    out_type=(jax.ShapeDtypeStruct((N,), jnp.float32),
              jax.ShapeDtypeStruct((N,), jnp.float32)),
    mesh=_MESH,
    scratch_types=[
        pltpu.VMEM_SHARED((N,), jnp.float32),
        pltpu.VMEM((EPT,), jnp.int32),
        pltpu.VMEM((EPT,), jnp.float32),
        pltpu.VMEM((_ZROWS,), jnp.float32),
    ],
)
def _deg_kernel(src_hbm, dst_hbm, do_hbm, di_hbm, acc, idx_v, ones_v, zline_v):
    c = lax.axis_index("c")
    s = lax.axis_index("s")

    @pl.loop(0, EPT // 16)
    def _(i):
        ones_v[pl.ds(i * 16, 16)] = jnp.ones((16,), jnp.float32)

    @pl.when(s < N // _ZROWS)
    def _():
        @pl.loop(0, _ZROWS // 16)
        def _(i):
            zline_v[pl.ds(i * 16, 16)] = jnp.zeros((16,), jnp.float32)
        pltpu.sync_copy(zline_v, acc.at[pl.ds(s * _ZROWS, _ZROWS)])

    plsc.subcore_barrier()

    @pl.when(c == 0)
    def _():
        pltpu.sync_copy(src_hbm.at[pl.ds(s * EPT, EPT)], idx_v)

    @pl.when(c == 1)
    def _():
        pltpu.sync_copy(dst_hbm.at[pl.ds(s * EPT, EPT)], idx_v)

    pltpu.sync_copy(ones_v, acc.at[idx_v], add=True)

    plsc.subcore_barrier()

    @pl.when(s < N // _ZROWS)
    def _():
        pltpu.sync_copy(acc.at[pl.ds(s * _ZROWS, _ZROWS)], zline_v)

        @pl.when(c == 0)
        def _():
            pltpu.sync_copy(zline_v, do_hbm.at[pl.ds(s * _ZROWS, _ZROWS)])

        @pl.when(c == 1)
        def _():
            pltpu.sync_copy(zline_v, di_hbm.at[pl.ds(s * _ZROWS, _ZROWS)])


_ORPB = 40
_ONB = N // _ORPB


@functools.partial(
    pl.kernel,
    out_type=(jax.ShapeDtypeStruct((N, DH), jnp.float32),
              jax.ShapeDtypeStruct((N, DH), jnp.float32)),
    mesh=_MESH,
    scratch_types=[
        pltpu.VMEM_SHARED((N, DH), jnp.float32),
        pltpu.VMEM((EPT,), jnp.int32),
        pltpu.VMEM((EPT,), jnp.int32),
        pltpu.VMEM((NBUF, GSZ, DH), jnp.float32),
        pltpu.VMEM((_ORPB, DH), jnp.float32),
        pltpu.SemaphoreType.DMA,
        pltpu.SemaphoreType.DMA,
        pltpu.SemaphoreType.DMA,
        pltpu.SemaphoreType.DMA,
        pltpu.SemaphoreType.DMA,
        pltpu.SemaphoreType.DMA,
        pltpu.SemaphoreType.DMA,
        pltpu.SemaphoreType.DMA,
    ],
)
def _agg_kernel(h0_hbm, h1_hbm, src_hbm, dst_hbm, out0_hbm, out1_hbm,
                acc, sidx_v, didx_v, gbuf_v, zbuf_v,
                g0, g1, g2, g3, s0, s1, s2, s3):
    c = lax.axis_index("c")
    s = lax.axis_index("s")
    gsems = (g0, g1, g2, g3)
    ssems = (s0, s1, s2, s3)

    @pl.loop(0, _ORPB * (DH // 16))
    def _(i):
        zbuf_v[i // (DH // 16), pl.ds((i % (DH // 16)) * 16, 16)] = \
            jnp.zeros((16,), jnp.float32)

    @pl.loop(0, -(-_ONB // NS))
    def _(j):
        blk = s + j * NS

        @pl.when(blk < _ONB)
        def _():
            pltpu.sync_copy(zbuf_v, acc.at[pl.ds(blk * _ORPB, _ORPB)])

    pltpu.sync_copy(src_hbm.at[pl.ds(s * EPT, EPT)], sidx_v)
    pltpu.sync_copy(dst_hbm.at[pl.ds(s * EPT, EPT)], didx_v)

    plsc.subcore_barrier()

    def _gather_start(g, b):
        idx = sidx_v.at[pl.ds(g * GSZ, GSZ)]

        @pl.when(c == 0)
        def _():
            pltpu.make_async_copy(
                h0_hbm.at[idx], gbuf_v.at[b], gsems[b]).start()

        @pl.when(c == 1)
        def _():
            pltpu.make_async_copy(
                h1_hbm.at[idx], gbuf_v.at[b], gsems[b]).start()

    def _gather_wait(g, b):
        idx = sidx_v.at[pl.ds(g * GSZ, GSZ)]

        @pl.when(c == 0)
        def _():
            pltpu.make_async_copy(
                h0_hbm.at[idx], gbuf_v.at[b], gsems[b]).wait()

        @pl.when(c == 1)
        def _():
            pltpu.make_async_copy(
                h1_hbm.at[idx], gbuf_v.at[b], gsems[b]).wait()

    def _scatter_desc(g, b):
        return pltpu.make_async_copy(
            gbuf_v.at[b], acc.at[didx_v.at[pl.ds(g * GSZ, GSZ)]], ssems[b])

    for g in range(NBUF - 1):
        _gather_start(g, g)

    @pl.loop(0, -(-NG // NBUF))
    def _(t):
        for b in range(NBUF):
            g = t * NBUF + b

            @pl.when(g < NG)
            def _():
                _gather_wait(g, b)
                _scatter_desc(g, b).start(add=True)

                nxt = g + NBUF - 1
                b2 = (b + NBUF - 1) % NBUF

                @pl.when(nxt < NG)
                def _():
                    @pl.when(g >= 1)
                    def _():
                        _scatter_desc(g - 1, b2).wait()

                    _gather_start(nxt, b2)

    for k in range(NG - NBUF, NG):
        _scatter_desc(k, k % NBUF).wait()

    plsc.subcore_barrier()

    @pl.loop(0, -(-_ONB // NS))
    def _(j):
        blk = s + j * NS

        @pl.when(blk < _ONB)
        def _():
            base = blk * _ORPB
            pltpu.sync_copy(acc.at[pl.ds(base, _ORPB)], zbuf_v)

            @pl.when(c == 0)
            def _():
                pltpu.sync_copy(zbuf_v, out0_hbm.at[pl.ds(base, _ORPB)])

            @pl.when(c == 1)
            def _():
                pltpu.sync_copy(zbuf_v, out1_hbm.at[pl.ds(base, _ORPB)])


_BLK = 1000
_GRID = N // _BLK


def _norm(deg):
    return jnp.where(deg > 0, lax.rsqrt(jnp.maximum(deg, 1e-12)), 0.0)


def _mm1_body(x_ref, do_ref, w_ref, o0_ref, o1_ref):
    ns = _norm(do_ref[...])
    h = jnp.dot(x_ref[...] * ns, w_ref[...],
                preferred_element_type=jnp.float32)
    o0_ref[...] = h[:, :DH]
    o1_ref[...] = h[:, DH:]


_mm1 = pl.pallas_call(
    _mm1_body,
    grid=(_GRID,),
    in_specs=[
        pl.BlockSpec((_BLK, D), lambda i: (i, 0)),
        pl.BlockSpec((_BLK, 1), lambda i: (i, 0)),
        pl.BlockSpec((D, D), lambda i: (0, 0)),
    ],
    out_specs=[pl.BlockSpec((_BLK, DH), lambda i: (i, 0)),
               pl.BlockSpec((_BLK, DH), lambda i: (i, 0))],
    out_shape=(jax.ShapeDtypeStruct((N, DH), jnp.float32),
               jax.ShapeDtypeStruct((N, DH), jnp.float32)),
)


def _mid_body(a0_ref, a1_ref, di_ref, do_ref, b_ref, w_ref, o0_ref, o1_ref):
    nd = _norm(di_ref[...])
    ns = _norm(do_ref[...])
    t0 = jnp.maximum(a0_ref[...] * nd + b_ref[0, :DH], 0.0) * ns
    t1 = jnp.maximum(a1_ref[...] * nd + b_ref[0, DH:], 0.0) * ns
    h = (jnp.dot(t0, w_ref[:DH, :], preferred_element_type=jnp.float32)
         + jnp.dot(t1, w_ref[DH:, :], preferred_element_type=jnp.float32))
    o0_ref[...] = h[:, :DH]
    o1_ref[...] = h[:, DH:]


_mid = pl.pallas_call(
    _mid_body,
    grid=(_GRID,),
    in_specs=[
        pl.BlockSpec((_BLK, DH), lambda i: (i, 0)),
        pl.BlockSpec((_BLK, DH), lambda i: (i, 0)),
        pl.BlockSpec((_BLK, 1), lambda i: (i, 0)),
        pl.BlockSpec((_BLK, 1), lambda i: (i, 0)),
        pl.BlockSpec((1, D), lambda i: (0, 0)),
        pl.BlockSpec((D, D), lambda i: (0, 0)),
    ],
    out_specs=[pl.BlockSpec((_BLK, DH), lambda i: (i, 0)),
               pl.BlockSpec((_BLK, DH), lambda i: (i, 0))],
    out_shape=(jax.ShapeDtypeStruct((N, DH), jnp.float32),
               jax.ShapeDtypeStruct((N, DH), jnp.float32)),
)


def _fin_body(a0_ref, a1_ref, di_ref, b_ref, o_ref):
    nd = _norm(di_ref[...])
    t0 = jnp.maximum(a0_ref[...] * nd + b_ref[0, :DH], 0.0)
    t1 = jnp.maximum(a1_ref[...] * nd + b_ref[0, DH:], 0.0)
    o_ref[...] = jnp.concatenate([t0, t1], axis=1)


_fin = pl.pallas_call(
    _fin_body,
    grid=(_GRID,),
    in_specs=[
        pl.BlockSpec((_BLK, DH), lambda i: (i, 0)),
        pl.BlockSpec((_BLK, DH), lambda i: (i, 0)),
        pl.BlockSpec((_BLK, 1), lambda i: (i, 0)),
        pl.BlockSpec((1, D), lambda i: (0, 0)),
    ],
    out_specs=pl.BlockSpec((_BLK, D), lambda i: (i, 0)),
    out_shape=jax.ShapeDtypeStruct((N, D), jnp.float32),
)


def kernel(feat, edge_index, W1, b1, W2, b2):
    ei = edge_index.astype(jnp.int32)
    src, dst = ei[0], ei[1]
    deg_out, deg_in = _deg_kernel(src, dst)
    do = deg_out.reshape(N, 1)
    di = deg_in.reshape(N, 1)
    h0, h1 = _mm1(feat, do, W1)
    a0, a1 = _agg_kernel(h0, h1, src, dst)
    h0, h1 = _mid(a0, a1, di, do, b1.reshape(1, D), W2)
    a0, a1 = _agg_kernel(h0, h1, src, dst)
    return _fin(a0, a1, di, b2.reshape(1, D))


# speedup vs baseline: 9.6537x; 1.0455x over previous
import functools

import jax
import jax.numpy as jnp
from jax import lax
from jax.experimental import pallas as pl
from jax.experimental.pallas import tpu as pltpu
from jax.experimental.pallas import tpu_sc as plsc

N = 10000
E = 160000
D = 256
DH = 128

NC = 2
NS = 16
EPT = E // NS
GSZ = 40
NG = EPT // GSZ
NBUF = 4

_MESH = plsc.VectorSubcoreMesh(core_axis_name="c", subcore_axis_name="s")


_ZROWS = 2000


@functools.partial(
    pl.kernel,
    out_type=(jax.ShapeDtypeStruct((N,), jnp.float32),
              jax.ShapeDtypeStruct((N,), jnp.float32)),
    mesh=_MESH,
    scratch_types=[
        pltpu.VMEM_SHARED((N,), jnp.float32),
        pltpu.VMEM((EPT,), jnp.int32),
        pltpu.VMEM((EPT,), jnp.float32),
        pltpu.VMEM((_ZROWS,), jnp.float32),
    ],
)
def _deg_kernel(src_hbm, dst_hbm, do_hbm, di_hbm, acc, idx_v, ones_v, zline_v):
    c = lax.axis_index("c")
    s = lax.axis_index("s")

    @pl.loop(0, EPT // 16)
    def _(i):
        ones_v[pl.ds(i * 16, 16)] = jnp.ones((16,), jnp.float32)

    @pl.when(s < N // _ZROWS)
    def _():
        @pl.loop(0, _ZROWS // 16)
        def _(i):
            zline_v[pl.ds(i * 16, 16)] = jnp.zeros((16,), jnp.float32)
        pltpu.sync_copy(zline_v, acc.at[pl.ds(s * _ZROWS, _ZROWS)])

    plsc.subcore_barrier()

    @pl.when(c == 0)
    def _():
        pltpu.sync_copy(src_hbm.at[pl.ds(s * EPT, EPT)], idx_v)

    @pl.when(c == 1)
    def _():
        pltpu.sync_copy(dst_hbm.at[pl.ds(s * EPT, EPT)], idx_v)

    pltpu.sync_copy(ones_v, acc.at[idx_v], add=True)

    plsc.subcore_barrier()

    @pl.when(s < N // _ZROWS)
    def _():
        pltpu.sync_copy(acc.at[pl.ds(s * _ZROWS, _ZROWS)], zline_v)

        @pl.when(c == 0)
        def _():
            pltpu.sync_copy(zline_v, do_hbm.at[pl.ds(s * _ZROWS, _ZROWS)])

        @pl.when(c == 1)
        def _():
            pltpu.sync_copy(zline_v, di_hbm.at[pl.ds(s * _ZROWS, _ZROWS)])


_ORPB = 40
_ONB = N // _ORPB


@functools.partial(
    pl.kernel,
    out_type=(jax.ShapeDtypeStruct((N, DH), jnp.float32),
              jax.ShapeDtypeStruct((N, DH), jnp.float32)),
    mesh=_MESH,
    scratch_types=[
        pltpu.VMEM_SHARED((N, DH), jnp.float32),
        pltpu.VMEM((EPT,), jnp.int32),
        pltpu.VMEM((EPT,), jnp.int32),
        pltpu.VMEM((NBUF, GSZ, DH), jnp.float32),
        pltpu.VMEM((2, _ORPB, DH), jnp.float32),
        pltpu.SemaphoreType.DMA,
        pltpu.SemaphoreType.DMA,
        pltpu.SemaphoreType.DMA,
        pltpu.SemaphoreType.DMA,
        pltpu.SemaphoreType.DMA,
        pltpu.SemaphoreType.DMA,
        pltpu.SemaphoreType.DMA,
        pltpu.SemaphoreType.DMA,
        pltpu.SemaphoreType.DMA,
        pltpu.SemaphoreType.DMA,
        pltpu.SemaphoreType.DMA,
    ],
)
def _agg_kernel(h0_hbm, h1_hbm, src_hbm, dst_hbm, out0_hbm, out1_hbm,
                acc, sidx_v, didx_v, gbuf_v, zbuf_v,
                g0, g1, g2, g3, s0, s1, s2, s3, psem0, psem1, zsem):
    c = lax.axis_index("c")
    s = lax.axis_index("s")
    gsems = (g0, g1, g2, g3)
    ssems = (s0, s1, s2, s3)

    p0 = pltpu.make_async_copy(src_hbm.at[pl.ds(s * EPT, EPT)], sidx_v, psem0)
    p1 = pltpu.make_async_copy(dst_hbm.at[pl.ds(s * EPT, EPT)], didx_v, psem1)
    p0.start()
    p1.start()

    zb0 = zbuf_v.at[0]

    @pl.loop(0, _ORPB * (DH // 16))
    def _(i):
        zb0[i // (DH // 16), pl.ds((i % (DH // 16)) * 16, 16)] = \
            jnp.zeros((16,), jnp.float32)

    @pl.loop(0, -(-_ONB // NS))
    def _(j):
        blk = s + j * NS

        @pl.when(blk < _ONB)
        def _():
            pltpu.make_async_copy(
                zb0, acc.at[pl.ds(blk * _ORPB, _ORPB)], zsem).start()

    @pl.loop(0, -(-_ONB // NS))
    def _(j):
        blk = s + j * NS

        @pl.when(blk < _ONB)
        def _():
            pltpu.make_async_copy(
                zb0, acc.at[pl.ds(blk * _ORPB, _ORPB)], zsem).wait()

    p0.wait()
    p1.wait()

    plsc.subcore_barrier()

    def _gather_start(g, b):
        idx = sidx_v.at[pl.ds(g * GSZ, GSZ)]

        @pl.when(c == 0)
        def _():
            pltpu.make_async_copy(
                h0_hbm.at[idx], gbuf_v.at[b], gsems[b]).start()

        @pl.when(c == 1)
        def _():
            pltpu.make_async_copy(
                h1_hbm.at[idx], gbuf_v.at[b], gsems[b]).start()

    def _gather_wait(g, b):
        idx = sidx_v.at[pl.ds(g * GSZ, GSZ)]

        @pl.when(c == 0)
        def _():
            pltpu.make_async_copy(
                h0_hbm.at[idx], gbuf_v.at[b], gsems[b]).wait()

        @pl.when(c == 1)
        def _():
            pltpu.make_async_copy(
                h1_hbm.at[idx], gbuf_v.at[b], gsems[b]).wait()

    def _scatter_desc(g, b):
        return pltpu.make_async_copy(
            gbuf_v.at[b], acc.at[didx_v.at[pl.ds(g * GSZ, GSZ)]], ssems[b])

    for g in range(NBUF - 1):
        _gather_start(g, g)

    @pl.loop(0, -(-NG // NBUF))
    def _(t):
        for b in range(NBUF):
            g = t * NBUF + b

            @pl.when(g < NG)
            def _():
                _gather_wait(g, b)
                _scatter_desc(g, b).start(add=True)

                nxt = g + NBUF - 1
                b2 = (b + NBUF - 1) % NBUF

                @pl.when(nxt < NG)
                def _():
                    @pl.when(g >= 1)
                    def _():
                        _scatter_desc(g - 1, b2).wait()

                    _gather_start(nxt, b2)

    for k in range(NG - NBUF, NG):
        _scatter_desc(k, k % NBUF).wait()

    plsc.subcore_barrier()

    def _out_dma(blk, b):
        base = blk * _ORPB

        @pl.when(c == 0)
        def _():
            pltpu.make_async_copy(
                zbuf_v.at[b], out0_hbm.at[pl.ds(base, _ORPB)],
                (psem0, psem1)[b]).start()

        @pl.when(c == 1)
        def _():
            pltpu.make_async_copy(
                zbuf_v.at[b], out1_hbm.at[pl.ds(base, _ORPB)],
                (psem0, psem1)[b]).start()

    def _out_wait(blk, b):
        base = blk * _ORPB

        @pl.when(c == 0)
        def _():
            pltpu.make_async_copy(
                zbuf_v.at[b], out0_hbm.at[pl.ds(base, _ORPB)],
                (psem0, psem1)[b]).wait()

        @pl.when(c == 1)
        def _():
            pltpu.make_async_copy(
                zbuf_v.at[b], out1_hbm.at[pl.ds(base, _ORPB)],
                (psem0, psem1)[b]).wait()

    @pl.loop(0, -(-_ONB // NS) // 2 + 1)
    def _(t):
        for b in range(2):
            j = t * 2 + b
            blk = s + j * NS

            @pl.when(blk < _ONB)
            def _():
                @pl.when(j >= 2)
                def _():
                    _out_wait(blk - 2 * NS, b)

                pltpu.sync_copy(acc.at[pl.ds(blk * _ORPB, _ORPB)],
                                zbuf_v.at[b])
                _out_dma(blk, b)

    for b in range(2):
        last_j = lax.div(_ONB - 1 - s, NS)
        blk = s + last_j * NS
        blk_b = jnp.where(lax.rem(last_j, 2) == b, blk,
                          blk - NS)

        @pl.when(blk_b >= 0)
        def _():
            _out_wait(blk_b, b)


_BLK = 1000
_GRID = N // _BLK


def _norm(deg):
    return jnp.where(deg > 0, lax.rsqrt(jnp.maximum(deg, 1e-12)), 0.0)


def _mm1_body(x_ref, do_ref, w_ref, o0_ref, o1_ref):
    ns = _norm(do_ref[...])
    h = jnp.dot(x_ref[...] * ns, w_ref[...],
                preferred_element_type=jnp.float32)
    o0_ref[...] = h[:, :DH]
    o1_ref[...] = h[:, DH:]


_mm1 = pl.pallas_call(
    _mm1_body,
    grid=(_GRID,),
    in_specs=[
        pl.BlockSpec((_BLK, D), lambda i: (i, 0)),
        pl.BlockSpec((_BLK, 1), lambda i: (i, 0)),
        pl.BlockSpec((D, D), lambda i: (0, 0)),
    ],
    out_specs=[pl.BlockSpec((_BLK, DH), lambda i: (i, 0)),
               pl.BlockSpec((_BLK, DH), lambda i: (i, 0))],
    out_shape=(jax.ShapeDtypeStruct((N, DH), jnp.float32),
               jax.ShapeDtypeStruct((N, DH), jnp.float32)),
)


def _mid_body(a0_ref, a1_ref, di_ref, do_ref, b_ref, w_ref, o0_ref, o1_ref):
    nd = _norm(di_ref[...])
    ns = _norm(do_ref[...])
    t0 = jnp.maximum(a0_ref[...] * nd + b_ref[0, :DH], 0.0) * ns
    t1 = jnp.maximum(a1_ref[...] * nd + b_ref[0, DH:], 0.0) * ns
    h = (jnp.dot(t0, w_ref[:DH, :], preferred_element_type=jnp.float32)
         + jnp.dot(t1, w_ref[DH:, :], preferred_element_type=jnp.float32))
    o0_ref[...] = h[:, :DH]
    o1_ref[...] = h[:, DH:]


_mid = pl.pallas_call(
    _mid_body,
    grid=(_GRID,),
    in_specs=[
        pl.BlockSpec((_BLK, DH), lambda i: (i, 0)),
        pl.BlockSpec((_BLK, DH), lambda i: (i, 0)),
        pl.BlockSpec((_BLK, 1), lambda i: (i, 0)),
        pl.BlockSpec((_BLK, 1), lambda i: (i, 0)),
        pl.BlockSpec((1, D), lambda i: (0, 0)),
        pl.BlockSpec((D, D), lambda i: (0, 0)),
    ],
    out_specs=[pl.BlockSpec((_BLK, DH), lambda i: (i, 0)),
               pl.BlockSpec((_BLK, DH), lambda i: (i, 0))],
    out_shape=(jax.ShapeDtypeStruct((N, DH), jnp.float32),
               jax.ShapeDtypeStruct((N, DH), jnp.float32)),
)


def _fin_body(a0_ref, a1_ref, di_ref, b_ref, o_ref):
    nd = _norm(di_ref[...])
    t0 = jnp.maximum(a0_ref[...] * nd + b_ref[0, :DH], 0.0)
    t1 = jnp.maximum(a1_ref[...] * nd + b_ref[0, DH:], 0.0)
    o_ref[...] = jnp.concatenate([t0, t1], axis=1)


_fin = pl.pallas_call(
    _fin_body,
    grid=(_GRID,),
    in_specs=[
        pl.BlockSpec((_BLK, DH), lambda i: (i, 0)),
        pl.BlockSpec((_BLK, DH), lambda i: (i, 0)),
        pl.BlockSpec((_BLK, 1), lambda i: (i, 0)),
        pl.BlockSpec((1, D), lambda i: (0, 0)),
    ],
    out_specs=pl.BlockSpec((_BLK, D), lambda i: (i, 0)),
    out_shape=jax.ShapeDtypeStruct((N, D), jnp.float32),
)


def kernel(feat, edge_index, W1, b1, W2, b2):
    ei = edge_index.astype(jnp.int32)
    src, dst = ei[0], ei[1]
    deg_out, deg_in = _deg_kernel(src, dst)
    do = deg_out.reshape(N, 1)
    di = deg_in.reshape(N, 1)
    h0, h1 = _mm1(feat, do, W1)
    a0, a1 = _agg_kernel(h0, h1, src, dst)
    h0, h1 = _mid(a0, a1, di, do, b1.reshape(1, D), W2)
    a0, a1 = _agg_kernel(h0, h1, src, dst)
    return _fin(a0, a1, di, b2.reshape(1, D))


# speedup vs baseline: 10.1358x; 1.0499x over previous
import functools

import jax
import jax.numpy as jnp
from jax import lax
from jax.experimental import pallas as pl
from jax.experimental.pallas import tpu as pltpu
from jax.experimental.pallas import tpu_sc as plsc

N = 10000
E = 160000
D = 256
DH = 128

NC = 2
NS = 16
EPT = E // NS
GSZ = 40
NG = EPT // GSZ
NBUF = 9

_MESH = plsc.VectorSubcoreMesh(core_axis_name="c", subcore_axis_name="s")


_ZROWS = 2000


@functools.partial(
    pl.kernel,
    out_type=(jax.ShapeDtypeStruct((N,), jnp.float32),
              jax.ShapeDtypeStruct((N,), jnp.float32)),
    mesh=_MESH,
    scratch_types=[
        pltpu.VMEM_SHARED((N,), jnp.float32),
        pltpu.VMEM((EPT,), jnp.int32),
        pltpu.VMEM((EPT,), jnp.float32),
        pltpu.VMEM((_ZROWS,), jnp.float32),
    ],
)
def _deg_kernel(src_hbm, dst_hbm, do_hbm, di_hbm, acc, idx_v, ones_v, zline_v):
    c = lax.axis_index("c")
    s = lax.axis_index("s")

    @pl.loop(0, EPT // 16)
    def _(i):
        ones_v[pl.ds(i * 16, 16)] = jnp.ones((16,), jnp.float32)

    @pl.when(s < N // _ZROWS)
    def _():
        @pl.loop(0, _ZROWS // 16)
        def _(i):
            zline_v[pl.ds(i * 16, 16)] = jnp.zeros((16,), jnp.float32)
        pltpu.sync_copy(zline_v, acc.at[pl.ds(s * _ZROWS, _ZROWS)])

    plsc.subcore_barrier()

    @pl.when(c == 0)
    def _():
        pltpu.sync_copy(src_hbm.at[pl.ds(s * EPT, EPT)], idx_v)

    @pl.when(c == 1)
    def _():
        pltpu.sync_copy(dst_hbm.at[pl.ds(s * EPT, EPT)], idx_v)

    pltpu.sync_copy(ones_v, acc.at[idx_v], add=True)

    plsc.subcore_barrier()

    @pl.when(s < N // _ZROWS)
    def _():
        pltpu.sync_copy(acc.at[pl.ds(s * _ZROWS, _ZROWS)], zline_v)

        @pl.when(c == 0)
        def _():
            pltpu.sync_copy(zline_v, do_hbm.at[pl.ds(s * _ZROWS, _ZROWS)])

        @pl.when(c == 1)
        def _():
            pltpu.sync_copy(zline_v, di_hbm.at[pl.ds(s * _ZROWS, _ZROWS)])


_ORPB = GSZ
_ONB = N // _ORPB


def _sems(n):
    return [pltpu.SemaphoreType.DMA] * n


@functools.partial(
    pl.kernel,
    out_type=(jax.ShapeDtypeStruct((N, DH), jnp.float32),
              jax.ShapeDtypeStruct((N, DH), jnp.float32)),
    mesh=_MESH,
    scratch_types=[
        pltpu.VMEM_SHARED((N, DH), jnp.float32),
        pltpu.VMEM((NBUF, GSZ), jnp.int32),
        pltpu.VMEM((NBUF, GSZ), jnp.int32),
        pltpu.VMEM((NBUF, GSZ, DH), jnp.float32),
    ] + _sems(3 * NBUF + 3),
)
def _agg_kernel(h0_hbm, h1_hbm, src_hbm, dst_hbm, out0_hbm, out1_hbm,
                acc, sidx_v, didx_v, gbuf_v, *sems):
    c = lax.axis_index("c")
    s = lax.axis_index("s")
    isems = sems[0:NBUF]
    gsems = sems[NBUF:2 * NBUF]
    ssems = sems[2 * NBUF:3 * NBUF]
    zsem = sems[3 * NBUF]
    osems = sems[3 * NBUF + 1:3 * NBUF + 3]

    zb0 = gbuf_v.at[0]

    @pl.loop(0, GSZ * (DH // 16))
    def _(i):
        zb0[i // (DH // 16), pl.ds((i % (DH // 16)) * 16, 16)] = \
            jnp.zeros((16,), jnp.float32)

    @pl.loop(0, -(-_ONB // NS))
    def _(j):
        blk = s + j * NS

        @pl.when(blk < _ONB)
        def _():
            pltpu.make_async_copy(
                zb0, acc.at[pl.ds(blk * _ORPB, _ORPB)], zsem).start()

    @pl.loop(0, -(-_ONB // NS))
    def _(j):
        blk = s + j * NS

        @pl.when(blk < _ONB)
        def _():
            pltpu.make_async_copy(
                zb0, acc.at[pl.ds(blk * _ORPB, _ORPB)], zsem).wait()

    plsc.subcore_barrier()

    def _idx_descs(g, b):
        base = s * EPT + g * GSZ
        return (pltpu.make_async_copy(src_hbm.at[pl.ds(base, GSZ)],
                                      sidx_v.at[b], isems[b]),
                pltpu.make_async_copy(dst_hbm.at[pl.ds(base, GSZ)],
                                      didx_v.at[b], isems[b]))

    def _idx_start(g, b):
        d0, d1 = _idx_descs(g, b)
        d0.start()
        d1.start()

    def _idx_wait(g, b):
        d0, d1 = _idx_descs(g, b)
        d0.wait()
        d1.wait()

    def _gather_start(g, b):
        @pl.when(c == 0)
        def _():
            pltpu.make_async_copy(
                h0_hbm.at[sidx_v.at[b]], gbuf_v.at[b], gsems[b]).start()

        @pl.when(c == 1)
        def _():
            pltpu.make_async_copy(
                h1_hbm.at[sidx_v.at[b]], gbuf_v.at[b], gsems[b]).start()

    def _gather_wait(g, b):
        @pl.when(c == 0)
        def _():
            pltpu.make_async_copy(
                h0_hbm.at[sidx_v.at[b]], gbuf_v.at[b], gsems[b]).wait()

        @pl.when(c == 1)
        def _():
            pltpu.make_async_copy(
                h1_hbm.at[sidx_v.at[b]], gbuf_v.at[b], gsems[b]).wait()

    def _scatter_desc(g, b):
        return pltpu.make_async_copy(
            gbuf_v.at[b], acc.at[didx_v.at[b]], ssems[b])

    for k in range(NBUF - 1):
        _idx_start(k, k)
    for k in range(NBUF - 2):
        _idx_wait(k, k)
        _gather_start(k, k)

    @pl.loop(0, -(-NG // NBUF))
    def _(t):
        for b in range(NBUF):
            g = t * NBUF + b

            @pl.when(g < NG)
            def _():
                _gather_wait(g, b)
                _scatter_desc(g, b).start(add=True)

                j1 = g + NBUF - 1
                b1 = (b + NBUF - 1) % NBUF

                @pl.when(j1 < NG)
                def _():
                    @pl.when(g >= 1)
                    def _():
                        _scatter_desc(g - 1, b1).wait()

                    _idx_start(j1, b1)

                j2 = g + NBUF - 2
                b2 = (b + NBUF - 2) % NBUF

                @pl.when(j2 < NG)
                def _():
                    _idx_wait(j2, b2)
                    _gather_start(j2, b2)

    for k in range(NG - NBUF, NG):
        _scatter_desc(k, k % NBUF).wait()

    plsc.subcore_barrier()

    def _out_desc(blk, b):
        base = blk * _ORPB
        dst = (out0_hbm, out1_hbm)

        @pl.when(c == 0)
        def _():
            pltpu.make_async_copy(
                gbuf_v.at[b], out0_hbm.at[pl.ds(base, _ORPB)],
                osems[b]).start()

        @pl.when(c == 1)
        def _():
            pltpu.make_async_copy(
                gbuf_v.at[b], out1_hbm.at[pl.ds(base, _ORPB)],
                osems[b]).start()

    def _out_wait(blk, b):
        base = blk * _ORPB

        @pl.when(c == 0)
        def _():
            pltpu.make_async_copy(
                gbuf_v.at[b], out0_hbm.at[pl.ds(base, _ORPB)],
                osems[b]).wait()

        @pl.when(c == 1)
        def _():
            pltpu.make_async_copy(
                gbuf_v.at[b], out1_hbm.at[pl.ds(base, _ORPB)],
                osems[b]).wait()

    @pl.loop(0, -(-_ONB // NS) // 2 + 1)
    def _(t):
        for b in range(2):
            j = t * 2 + b
            blk = s + j * NS

            @pl.when(blk < _ONB)
            def _():
                @pl.when(j >= 2)
                def _():
                    _out_wait(blk - 2 * NS, b)

                pltpu.sync_copy(acc.at[pl.ds(blk * _ORPB, _ORPB)],
                                gbuf_v.at[b])
                _out_dma_started = _out_desc(blk, b)

    for b in range(2):
        last_j = lax.div(_ONB - 1 - s, NS)
        blk = s + last_j * NS
        blk_b = jnp.where(lax.rem(last_j, 2) == b, blk, blk - NS)

        @pl.when(blk_b >= 0)
        def _():
            _out_wait(blk_b, b)


_BLK = 1000
_GRID = N // _BLK


def _norm(deg):
    return jnp.where(deg > 0, lax.rsqrt(jnp.maximum(deg, 1e-12)), 0.0)


def _mm1_body(x_ref, do_ref, w_ref, o0_ref, o1_ref):
    ns = _norm(do_ref[...])
    h = jnp.dot(x_ref[...] * ns, w_ref[...],
                preferred_element_type=jnp.float32)
    o0_ref[...] = h[:, :DH]
    o1_ref[...] = h[:, DH:]


_mm1 = pl.pallas_call(
    _mm1_body,
    grid=(_GRID,),
    in_specs=[
        pl.BlockSpec((_BLK, D), lambda i: (i, 0)),
        pl.BlockSpec((_BLK, 1), lambda i: (i, 0)),
        pl.BlockSpec((D, D), lambda i: (0, 0)),
    ],
    out_specs=[pl.BlockSpec((_BLK, DH), lambda i: (i, 0)),
               pl.BlockSpec((_BLK, DH), lambda i: (i, 0))],
    out_shape=(jax.ShapeDtypeStruct((N, DH), jnp.float32),
               jax.ShapeDtypeStruct((N, DH), jnp.float32)),
)


def _mid_body(a0_ref, a1_ref, di_ref, do_ref, b_ref, w_ref, o0_ref, o1_ref):
    nd = _norm(di_ref[...])
    ns = _norm(do_ref[...])
    t0 = jnp.maximum(a0_ref[...] * nd + b_ref[0, :DH], 0.0) * ns
    t1 = jnp.maximum(a1_ref[...] * nd + b_ref[0, DH:], 0.0) * ns
    h = (jnp.dot(t0, w_ref[:DH, :], preferred_element_type=jnp.float32)
         + jnp.dot(t1, w_ref[DH:, :], preferred_element_type=jnp.float32))
    o0_ref[...] = h[:, :DH]
    o1_ref[...] = h[:, DH:]


_mid = pl.pallas_call(
    _mid_body,
    grid=(_GRID,),
    in_specs=[
        pl.BlockSpec((_BLK, DH), lambda i: (i, 0)),
        pl.BlockSpec((_BLK, DH), lambda i: (i, 0)),
        pl.BlockSpec((_BLK, 1), lambda i: (i, 0)),
        pl.BlockSpec((_BLK, 1), lambda i: (i, 0)),
        pl.BlockSpec((1, D), lambda i: (0, 0)),
        pl.BlockSpec((D, D), lambda i: (0, 0)),
    ],
    out_specs=[pl.BlockSpec((_BLK, DH), lambda i: (i, 0)),
               pl.BlockSpec((_BLK, DH), lambda i: (i, 0))],
    out_shape=(jax.ShapeDtypeStruct((N, DH), jnp.float32),
               jax.ShapeDtypeStruct((N, DH), jnp.float32)),
)


def _fin_body(a0_ref, a1_ref, di_ref, b_ref, o_ref):
    nd = _norm(di_ref[...])
    t0 = jnp.maximum(a0_ref[...] * nd + b_ref[0, :DH], 0.0)
    t1 = jnp.maximum(a1_ref[...] * nd + b_ref[0, DH:], 0.0)
    o_ref[...] = jnp.concatenate([t0, t1], axis=1)


_fin = pl.pallas_call(
    _fin_body,
    grid=(_GRID,),
    in_specs=[
        pl.BlockSpec((_BLK, DH), lambda i: (i, 0)),
        pl.BlockSpec((_BLK, DH), lambda i: (i, 0)),
        pl.BlockSpec((_BLK, 1), lambda i: (i, 0)),
        pl.BlockSpec((1, D), lambda i: (0, 0)),
    ],
    out_specs=pl.BlockSpec((_BLK, D), lambda i: (i, 0)),
    out_shape=jax.ShapeDtypeStruct((N, D), jnp.float32),
)


def kernel(feat, edge_index, W1, b1, W2, b2):
    ei = edge_index.astype(jnp.int32)
    src, dst = ei[0], ei[1]
    deg_out, deg_in = _deg_kernel(src, dst)
    do = deg_out.reshape(N, 1)
    di = deg_in.reshape(N, 1)
    h0, h1 = _mm1(feat, do, W1)
    a0, a1 = _agg_kernel(h0, h1, src, dst)
    h0, h1 = _mid(a0, a1, di, do, b1.reshape(1, D), W2)
    a0, a1 = _agg_kernel(h0, h1, src, dst)
    return _fin(a0, a1, di, b2.reshape(1, D))


# speedup vs baseline: 10.3400x; 1.0202x over previous
import functools

import jax
import jax.numpy as jnp
from jax import lax
from jax.experimental import pallas as pl
from jax.experimental.pallas import tpu as pltpu
from jax.experimental.pallas import tpu_sc as plsc

N = 10000
E = 160000
D = 256
DH = 128

NC = 2
NS = 16
EPT = E // NS
GSZ = 40
NG = EPT // GSZ
NBUF = 9

_MESH = plsc.VectorSubcoreMesh(core_axis_name="c", subcore_axis_name="s")


_ZROWS = 2000


@functools.partial(
    pl.kernel,
    out_type=(jax.ShapeDtypeStruct((N,), jnp.float32),
              jax.ShapeDtypeStruct((N,), jnp.float32)),
    mesh=_MESH,
    scratch_types=[
        pltpu.VMEM_SHARED((N,), jnp.float32),
        pltpu.VMEM((EPT,), jnp.int32),
        pltpu.VMEM((EPT,), jnp.float32),
        pltpu.VMEM((_ZROWS,), jnp.float32),
    ],
)
def _deg_kernel(ei_hbm, do_hbm, di_hbm, acc, idx_v, ones_v, zline_v):
    c = lax.axis_index("c")
    s = lax.axis_index("s")

    @pl.loop(0, EPT // 16)
    def _(i):
        ones_v[pl.ds(i * 16, 16)] = jnp.ones((16,), jnp.float32)

    @pl.when(s < N // _ZROWS)
    def _():
        @pl.loop(0, _ZROWS // 16)
        def _(i):
            zline_v[pl.ds(i * 16, 16)] = jnp.zeros((16,), jnp.float32)
        pltpu.sync_copy(zline_v, acc.at[pl.ds(s * _ZROWS, _ZROWS)])

    plsc.subcore_barrier()

    pltpu.sync_copy(ei_hbm.at[pl.ds(c * E + s * EPT, EPT)], idx_v)

    pltpu.sync_copy(ones_v, acc.at[idx_v], add=True)

    plsc.subcore_barrier()

    @pl.when(s < N // _ZROWS)
    def _():
        pltpu.sync_copy(acc.at[pl.ds(s * _ZROWS, _ZROWS)], zline_v)

        @pl.when(c == 0)
        def _():
            pltpu.sync_copy(zline_v, do_hbm.at[pl.ds(s * _ZROWS, _ZROWS)])

        @pl.when(c == 1)
        def _():
            pltpu.sync_copy(zline_v, di_hbm.at[pl.ds(s * _ZROWS, _ZROWS)])


_ORPB = GSZ
_ONB = N // _ORPB


def _sems(n):
    return [pltpu.SemaphoreType.DMA] * n


@functools.partial(
    pl.kernel,
    out_type=(jax.ShapeDtypeStruct((N, DH), jnp.float32),
              jax.ShapeDtypeStruct((N, DH), jnp.float32)),
    mesh=_MESH,
    scratch_types=[
        pltpu.VMEM_SHARED((N, DH), jnp.float32),
        pltpu.VMEM((NBUF, GSZ), jnp.int32),
        pltpu.VMEM((NBUF, GSZ), jnp.int32),
        pltpu.VMEM((NBUF, GSZ, DH), jnp.float32),
    ] + _sems(3 * NBUF + 3),
)
def _agg_kernel(h0_hbm, h1_hbm, ei_hbm, out0_hbm, out1_hbm,
                acc, sidx_v, didx_v, gbuf_v, *sems):
    c = lax.axis_index("c")
    s = lax.axis_index("s")
    isems = sems[0:NBUF]
    gsems = sems[NBUF:2 * NBUF]
    ssems = sems[2 * NBUF:3 * NBUF]
    zsem = sems[3 * NBUF]
    osems = sems[3 * NBUF + 1:3 * NBUF + 3]

    zb0 = gbuf_v.at[0]

    @pl.loop(0, GSZ * (DH // 16))
    def _(i):
        zb0[i // (DH // 16), pl.ds((i % (DH // 16)) * 16, 16)] = \
            jnp.zeros((16,), jnp.float32)

    @pl.loop(0, -(-_ONB // NS))
    def _(j):
        blk = s + j * NS

        @pl.when(blk < _ONB)
        def _():
            pltpu.make_async_copy(
                zb0, acc.at[pl.ds(blk * _ORPB, _ORPB)], zsem).start()

    @pl.loop(0, -(-_ONB // NS))
    def _(j):
        blk = s + j * NS

        @pl.when(blk < _ONB)
        def _():
            pltpu.make_async_copy(
                zb0, acc.at[pl.ds(blk * _ORPB, _ORPB)], zsem).wait()

    plsc.subcore_barrier()

    def _idx_descs(g, b):
        base = s * EPT + g * GSZ
        return (pltpu.make_async_copy(ei_hbm.at[pl.ds(base, GSZ)],
                                      sidx_v.at[b], isems[b]),
                pltpu.make_async_copy(ei_hbm.at[pl.ds(E + base, GSZ)],
                                      didx_v.at[b], isems[b]))

    def _idx_start(g, b):
        d0, d1 = _idx_descs(g, b)
        d0.start()
        d1.start()

    def _idx_wait(g, b):
        d0, d1 = _idx_descs(g, b)
        d0.wait()
        d1.wait()

    def _gather_start(g, b):
        @pl.when(c == 0)
        def _():
            pltpu.make_async_copy(
                h0_hbm.at[sidx_v.at[b]], gbuf_v.at[b], gsems[b]).start()

        @pl.when(c == 1)
        def _():
            pltpu.make_async_copy(
                h1_hbm.at[sidx_v.at[b]], gbuf_v.at[b], gsems[b]).start()

    def _gather_wait(g, b):
        @pl.when(c == 0)
        def _():
            pltpu.make_async_copy(
                h0_hbm.at[sidx_v.at[b]], gbuf_v.at[b], gsems[b]).wait()

        @pl.when(c == 1)
        def _():
            pltpu.make_async_copy(
                h1_hbm.at[sidx_v.at[b]], gbuf_v.at[b], gsems[b]).wait()

    def _scatter_desc(g, b):
        return pltpu.make_async_copy(
            gbuf_v.at[b], acc.at[didx_v.at[b]], ssems[b])

    for k in range(NBUF - 1):
        _idx_start(k, k)
    for k in range(NBUF - 2):
        _idx_wait(k, k)
        _gather_start(k, k)

    @pl.loop(0, -(-NG // NBUF))
    def _(t):
        for b in range(NBUF):
            g = t * NBUF + b

            @pl.when(g < NG)
            def _():
                _gather_wait(g, b)
                _scatter_desc(g, b).start(add=True)

                j1 = g + NBUF - 1
                b1 = (b + NBUF - 1) % NBUF

                @pl.when(j1 < NG)
                def _():
                    @pl.when(g >= 1)
                    def _():
                        _scatter_desc(g - 1, b1).wait()

                    _idx_start(j1, b1)

                j2 = g + NBUF - 2
                b2 = (b + NBUF - 2) % NBUF

                @pl.when(j2 < NG)
                def _():
                    _idx_wait(j2, b2)
                    _gather_start(j2, b2)

    for k in range(NG - NBUF, NG):
        _scatter_desc(k, k % NBUF).wait()

    plsc.subcore_barrier()

    def _out_desc(blk, b):
        base = blk * _ORPB
        dst = (out0_hbm, out1_hbm)

        @pl.when(c == 0)
        def _():
            pltpu.make_async_copy(
                gbuf_v.at[b], out0_hbm.at[pl.ds(base, _ORPB)],
                osems[b]).start()

        @pl.when(c == 1)
        def _():
            pltpu.make_async_copy(
                gbuf_v.at[b], out1_hbm.at[pl.ds(base, _ORPB)],
                osems[b]).start()

    def _out_wait(blk, b):
        base = blk * _ORPB

        @pl.when(c == 0)
        def _():
            pltpu.make_async_copy(
                gbuf_v.at[b], out0_hbm.at[pl.ds(base, _ORPB)],
                osems[b]).wait()

        @pl.when(c == 1)
        def _():
            pltpu.make_async_copy(
                gbuf_v.at[b], out1_hbm.at[pl.ds(base, _ORPB)],
                osems[b]).wait()

    @pl.loop(0, -(-_ONB // NS) // 2 + 1)
    def _(t):
        for b in range(2):
            j = t * 2 + b
            blk = s + j * NS

            @pl.when(blk < _ONB)
            def _():
                @pl.when(j >= 2)
                def _():
                    _out_wait(blk - 2 * NS, b)

                pltpu.sync_copy(acc.at[pl.ds(blk * _ORPB, _ORPB)],
                                gbuf_v.at[b])
                _out_dma_started = _out_desc(blk, b)

    for b in range(2):
        last_j = lax.div(_ONB - 1 - s, NS)
        blk = s + last_j * NS
        blk_b = jnp.where(lax.rem(last_j, 2) == b, blk, blk - NS)

        @pl.when(blk_b >= 0)
        def _():
            _out_wait(blk_b, b)


_BLK = 1000
_GRID = N // _BLK


def _norm(deg):
    return jnp.where(deg > 0, lax.rsqrt(jnp.maximum(deg, 1e-12)), 0.0)


def _mm1_body(x_ref, do_ref, w_ref, o0_ref, o1_ref):
    ns = _norm(do_ref[...])
    h = jnp.dot((x_ref[...] * ns).astype(jnp.bfloat16),
                w_ref[...].astype(jnp.bfloat16),
                preferred_element_type=jnp.float32)
    o0_ref[...] = h[:, :DH]
    o1_ref[...] = h[:, DH:]


_mm1 = pl.pallas_call(
    _mm1_body,
    grid=(_GRID,),
    in_specs=[
        pl.BlockSpec((_BLK, D), lambda i: (i, 0)),
        pl.BlockSpec((_BLK, 1), lambda i: (i, 0)),
        pl.BlockSpec((D, D), lambda i: (0, 0)),
    ],
    out_specs=[pl.BlockSpec((_BLK, DH), lambda i: (i, 0)),
               pl.BlockSpec((_BLK, DH), lambda i: (i, 0))],
    out_shape=(jax.ShapeDtypeStruct((N, DH), jnp.float32),
               jax.ShapeDtypeStruct((N, DH), jnp.float32)),
)


def _mid_body(a0_ref, a1_ref, di_ref, do_ref, b_ref, w_ref, o0_ref, o1_ref):
    nd = _norm(di_ref[...])
    ns = _norm(do_ref[...])
    t0 = jnp.maximum(a0_ref[...] * nd + b_ref[0, :DH], 0.0) * ns
    t1 = jnp.maximum(a1_ref[...] * nd + b_ref[0, DH:], 0.0) * ns
    wb = w_ref[...].astype(jnp.bfloat16)
    h = (jnp.dot(t0.astype(jnp.bfloat16), wb[:DH, :],
                 preferred_element_type=jnp.float32)
         + jnp.dot(t1.astype(jnp.bfloat16), wb[DH:, :],
                   preferred_element_type=jnp.float32))
    o0_ref[...] = h[:, :DH]
    o1_ref[...] = h[:, DH:]


_mid = pl.pallas_call(
    _mid_body,
    grid=(_GRID,),
    in_specs=[
        pl.BlockSpec((_BLK, DH), lambda i: (i, 0)),
        pl.BlockSpec((_BLK, DH), lambda i: (i, 0)),
        pl.BlockSpec((_BLK, 1), lambda i: (i, 0)),
        pl.BlockSpec((_BLK, 1), lambda i: (i, 0)),
        pl.BlockSpec((1, D), lambda i: (0, 0)),
        pl.BlockSpec((D, D), lambda i: (0, 0)),
    ],
    out_specs=[pl.BlockSpec((_BLK, DH), lambda i: (i, 0)),
               pl.BlockSpec((_BLK, DH), lambda i: (i, 0))],
    out_shape=(jax.ShapeDtypeStruct((N, DH), jnp.float32),
               jax.ShapeDtypeStruct((N, DH), jnp.float32)),
)


def _fin_body(a0_ref, a1_ref, di_ref, b_ref, o_ref):
    nd = _norm(di_ref[...])
    t0 = jnp.maximum(a0_ref[...] * nd + b_ref[0, :DH], 0.0)
    t1 = jnp.maximum(a1_ref[...] * nd + b_ref[0, DH:], 0.0)
    o_ref[...] = jnp.concatenate([t0, t1], axis=1)


_fin = pl.pallas_call(
    _fin_body,
    grid=(_GRID,),
    in_specs=[
        pl.BlockSpec((_BLK, DH), lambda i: (i, 0)),
        pl.BlockSpec((_BLK, DH), lambda i: (i, 0)),
        pl.BlockSpec((_BLK, 1), lambda i: (i, 0)),
        pl.BlockSpec((1, D), lambda i: (0, 0)),
    ],
    out_specs=pl.BlockSpec((_BLK, D), lambda i: (i, 0)),
    out_shape=jax.ShapeDtypeStruct((N, D), jnp.float32),
)


def kernel(feat, edge_index, W1, b1, W2, b2):
    ei = edge_index.astype(jnp.int32).reshape(2 * E)
    deg_out, deg_in = _deg_kernel(ei)
    do = deg_out.reshape(N, 1)
    di = deg_in.reshape(N, 1)
    h0, h1 = _mm1(feat, do, W1)
    a0, a1 = _agg_kernel(h0, h1, ei)
    h0, h1 = _mid(a0, a1, di, do, b1.reshape(1, D), W2)
    a0, a1 = _agg_kernel(h0, h1, ei)
    return _fin(a0, a1, di, b2.reshape(1, D))


# speedup vs baseline: 10.3809x; 1.0040x over previous
import functools

import jax
import jax.numpy as jnp
from jax import lax
from jax.experimental import pallas as pl
from jax.experimental.pallas import tpu as pltpu
from jax.experimental.pallas import tpu_sc as plsc

N = 10000
E = 160000
D = 256
DH = 128

NC = 2
NS = 16
EPT = E // NS
GSZ = 40
NG = EPT // GSZ
NBUF = 9

_MESH = plsc.VectorSubcoreMesh(core_axis_name="c", subcore_axis_name="s")


_ZROWS = 2000


@functools.partial(
    pl.kernel,
    out_type=(jax.ShapeDtypeStruct((N,), jnp.float32),
              jax.ShapeDtypeStruct((N,), jnp.float32)),
    mesh=_MESH,
    scratch_types=[
        pltpu.VMEM_SHARED((N,), jnp.float32),
        pltpu.VMEM((EPT,), jnp.int32),
        pltpu.VMEM((EPT,), jnp.float32),
        pltpu.VMEM((_ZROWS,), jnp.float32),
        pltpu.SemaphoreType.DMA,
    ],
)
def _deg_kernel(ei_hbm, do_hbm, di_hbm, acc, idx_v, ones_v, zline_v, isem):
    c = lax.axis_index("c")
    s = lax.axis_index("s")

    ipre = pltpu.make_async_copy(
        ei_hbm.at[pl.ds(c * E + s * EPT, EPT)], idx_v, isem)
    ipre.start()

    @pl.loop(0, EPT // 16)
    def _(i):
        ones_v[pl.ds(i * 16, 16)] = jnp.ones((16,), jnp.float32)

    @pl.when(s < N // _ZROWS)
    def _():
        @pl.loop(0, _ZROWS // 16)
        def _(i):
            zline_v[pl.ds(i * 16, 16)] = jnp.zeros((16,), jnp.float32)
        pltpu.sync_copy(zline_v, acc.at[pl.ds(s * _ZROWS, _ZROWS)])

    ipre.wait()
    plsc.subcore_barrier()

    pltpu.sync_copy(ones_v, acc.at[idx_v], add=True)

    plsc.subcore_barrier()

    @pl.when(s < N // _ZROWS)
    def _():
        pltpu.sync_copy(acc.at[pl.ds(s * _ZROWS, _ZROWS)], zline_v)

        @pl.when(c == 0)
        def _():
            pltpu.sync_copy(zline_v, do_hbm.at[pl.ds(s * _ZROWS, _ZROWS)])

        @pl.when(c == 1)
        def _():
            pltpu.sync_copy(zline_v, di_hbm.at[pl.ds(s * _ZROWS, _ZROWS)])


_ORPB = GSZ
_ONB = N // _ORPB


def _sems(n):
    return [pltpu.SemaphoreType.DMA] * n


@functools.partial(
    pl.kernel,
    out_type=(jax.ShapeDtypeStruct((N, DH), jnp.float32),
              jax.ShapeDtypeStruct((N, DH), jnp.float32)),
    mesh=_MESH,
    scratch_types=[
        pltpu.VMEM_SHARED((N, DH), jnp.float32),
        pltpu.VMEM((NBUF, GSZ), jnp.int32),
        pltpu.VMEM((NBUF, GSZ), jnp.int32),
        pltpu.VMEM((NBUF, GSZ, DH), jnp.float32),
    ] + _sems(3 * NBUF + 3),
)
def _agg_kernel(h0_hbm, h1_hbm, ei_hbm, out0_hbm, out1_hbm,
                acc, sidx_v, didx_v, gbuf_v, *sems):
    c = lax.axis_index("c")
    s = lax.axis_index("s")
    isems = sems[0:NBUF]
    gsems = sems[NBUF:2 * NBUF]
    ssems = sems[2 * NBUF:3 * NBUF]
    zsem = sems[3 * NBUF]
    osems = sems[3 * NBUF + 1:3 * NBUF + 3]

    zb0 = gbuf_v.at[0]

    @pl.loop(0, GSZ * (DH // 16))
    def _(i):
        zb0[i // (DH // 16), pl.ds((i % (DH // 16)) * 16, 16)] = \
            jnp.zeros((16,), jnp.float32)

    @pl.loop(0, -(-_ONB // NS))
    def _(j):
        blk = s + j * NS

        @pl.when(blk < _ONB)
        def _():
            pltpu.make_async_copy(
                zb0, acc.at[pl.ds(blk * _ORPB, _ORPB)], zsem).start()

    @pl.loop(0, -(-_ONB // NS))
    def _(j):
        blk = s + j * NS

        @pl.when(blk < _ONB)
        def _():
            pltpu.make_async_copy(
                zb0, acc.at[pl.ds(blk * _ORPB, _ORPB)], zsem).wait()

    plsc.subcore_barrier()

    def _idx_descs(g, b):
        base = s * EPT + g * GSZ
        return (pltpu.make_async_copy(ei_hbm.at[pl.ds(base, GSZ)],
                                      sidx_v.at[b], isems[b]),
                pltpu.make_async_copy(ei_hbm.at[pl.ds(E + base, GSZ)],
                                      didx_v.at[b], isems[b]))

    def _idx_start(g, b):
        d0, d1 = _idx_descs(g, b)
        d0.start()
        d1.start()

    def _idx_wait(g, b):
        d0, d1 = _idx_descs(g, b)
        d0.wait()
        d1.wait()

    def _gather_start(g, b):
        @pl.when(c == 0)
        def _():
            pltpu.make_async_copy(
                h0_hbm.at[sidx_v.at[b]], gbuf_v.at[b], gsems[b]).start()

        @pl.when(c == 1)
        def _():
            pltpu.make_async_copy(
                h1_hbm.at[sidx_v.at[b]], gbuf_v.at[b], gsems[b]).start()

    def _gather_wait(g, b):
        @pl.when(c == 0)
        def _():
            pltpu.make_async_copy(
                h0_hbm.at[sidx_v.at[b]], gbuf_v.at[b], gsems[b]).wait()

        @pl.when(c == 1)
        def _():
            pltpu.make_async_copy(
                h1_hbm.at[sidx_v.at[b]], gbuf_v.at[b], gsems[b]).wait()

    def _scatter_desc(g, b):
        return pltpu.make_async_copy(
            gbuf_v.at[b], acc.at[didx_v.at[b]], ssems[b])

    for k in range(NBUF - 1):
        _idx_start(k, k)
    for k in range(NBUF - 2):
        _idx_wait(k, k)
        _gather_start(k, k)

    @pl.loop(0, -(-NG // NBUF))
    def _(t):
        for b in range(NBUF):
            g = t * NBUF + b

            @pl.when(g < NG)
            def _():
                _gather_wait(g, b)
                _scatter_desc(g, b).start(add=True)

                j1 = g + NBUF - 1
                b1 = (b + NBUF - 1) % NBUF

                @pl.when(j1 < NG)
                def _():
                    @pl.when(g >= 1)
                    def _():
                        _scatter_desc(g - 1, b1).wait()

                    _idx_start(j1, b1)

                j2 = g + NBUF - 2
                b2 = (b + NBUF - 2) % NBUF

                @pl.when(j2 < NG)
                def _():
                    _idx_wait(j2, b2)
                    _gather_start(j2, b2)

    for k in range(NG - NBUF, NG):
        _scatter_desc(k, k % NBUF).wait()

    plsc.subcore_barrier()

    def _out_desc(blk, b):
        base = blk * _ORPB
        dst = (out0_hbm, out1_hbm)

        @pl.when(c == 0)
        def _():
            pltpu.make_async_copy(
                gbuf_v.at[b], out0_hbm.at[pl.ds(base, _ORPB)],
                osems[b]).start()

        @pl.when(c == 1)
        def _():
            pltpu.make_async_copy(
                gbuf_v.at[b], out1_hbm.at[pl.ds(base, _ORPB)],
                osems[b]).start()

    def _out_wait(blk, b):
        base = blk * _ORPB

        @pl.when(c == 0)
        def _():
            pltpu.make_async_copy(
                gbuf_v.at[b], out0_hbm.at[pl.ds(base, _ORPB)],
                osems[b]).wait()

        @pl.when(c == 1)
        def _():
            pltpu.make_async_copy(
                gbuf_v.at[b], out1_hbm.at[pl.ds(base, _ORPB)],
                osems[b]).wait()

    @pl.loop(0, -(-_ONB // NS) // 2 + 1)
    def _(t):
        for b in range(2):
            j = t * 2 + b
            blk = s + j * NS

            @pl.when(blk < _ONB)
            def _():
                @pl.when(j >= 2)
                def _():
                    _out_wait(blk - 2 * NS, b)

                pltpu.sync_copy(acc.at[pl.ds(blk * _ORPB, _ORPB)],
                                gbuf_v.at[b])
                _out_dma_started = _out_desc(blk, b)

    for b in range(2):
        last_j = lax.div(_ONB - 1 - s, NS)
        blk = s + last_j * NS
        blk_b = jnp.where(lax.rem(last_j, 2) == b, blk, blk - NS)

        @pl.when(blk_b >= 0)
        def _():
            _out_wait(blk_b, b)


_BLK = 1000
_GRID = N // _BLK


def _norm(deg):
    return jnp.where(deg > 0, lax.rsqrt(jnp.maximum(deg, 1e-12)), 0.0)


def _mm1_body(x_ref, do_ref, w_ref, o0_ref, o1_ref):
    ns = _norm(do_ref[...])
    h = jnp.dot((x_ref[...] * ns).astype(jnp.bfloat16),
                w_ref[...].astype(jnp.bfloat16),
                preferred_element_type=jnp.float32)
    o0_ref[...] = h[:, :DH]
    o1_ref[...] = h[:, DH:]


_mm1 = pl.pallas_call(
    _mm1_body,
    grid=(_GRID,),
    in_specs=[
        pl.BlockSpec((_BLK, D), lambda i: (i, 0)),
        pl.BlockSpec((_BLK, 1), lambda i: (i, 0)),
        pl.BlockSpec((D, D), lambda i: (0, 0)),
    ],
    out_specs=[pl.BlockSpec((_BLK, DH), lambda i: (i, 0)),
               pl.BlockSpec((_BLK, DH), lambda i: (i, 0))],
    out_shape=(jax.ShapeDtypeStruct((N, DH), jnp.float32),
               jax.ShapeDtypeStruct((N, DH), jnp.float32)),
)


def _mid_body(a0_ref, a1_ref, di_ref, do_ref, b_ref, w_ref, o0_ref, o1_ref):
    nd = _norm(di_ref[...])
    ns = _norm(do_ref[...])
    t0 = jnp.maximum(a0_ref[...] * nd + b_ref[0, :DH], 0.0) * ns
    t1 = jnp.maximum(a1_ref[...] * nd + b_ref[0, DH:], 0.0) * ns
    wb = w_ref[...].astype(jnp.bfloat16)
    h = (jnp.dot(t0.astype(jnp.bfloat16), wb[:DH, :],
                 preferred_element_type=jnp.float32)
         + jnp.dot(t1.astype(jnp.bfloat16), wb[DH:, :],
                   preferred_element_type=jnp.float32))
    o0_ref[...] = h[:, :DH]
    o1_ref[...] = h[:, DH:]


_mid = pl.pallas_call(
    _mid_body,
    grid=(_GRID,),
    in_specs=[
        pl.BlockSpec((_BLK, DH), lambda i: (i, 0)),
        pl.BlockSpec((_BLK, DH), lambda i: (i, 0)),
        pl.BlockSpec((_BLK, 1), lambda i: (i, 0)),
        pl.BlockSpec((_BLK, 1), lambda i: (i, 0)),
        pl.BlockSpec((1, D), lambda i: (0, 0)),
        pl.BlockSpec((D, D), lambda i: (0, 0)),
    ],
    out_specs=[pl.BlockSpec((_BLK, DH), lambda i: (i, 0)),
               pl.BlockSpec((_BLK, DH), lambda i: (i, 0))],
    out_shape=(jax.ShapeDtypeStruct((N, DH), jnp.float32),
               jax.ShapeDtypeStruct((N, DH), jnp.float32)),
)


def _fin_body(a0_ref, a1_ref, di_ref, b_ref, o_ref):
    nd = _norm(di_ref[...])
    t0 = jnp.maximum(a0_ref[...] * nd + b_ref[0, :DH], 0.0)
    t1 = jnp.maximum(a1_ref[...] * nd + b_ref[0, DH:], 0.0)
    o_ref[...] = jnp.concatenate([t0, t1], axis=1)


_fin = pl.pallas_call(
    _fin_body,
    grid=(_GRID,),
    in_specs=[
        pl.BlockSpec((_BLK, DH), lambda i: (i, 0)),
        pl.BlockSpec((_BLK, DH), lambda i: (i, 0)),
        pl.BlockSpec((_BLK, 1), lambda i: (i, 0)),
        pl.BlockSpec((1, D), lambda i: (0, 0)),
    ],
    out_specs=pl.BlockSpec((_BLK, D), lambda i: (i, 0)),
    out_shape=jax.ShapeDtypeStruct((N, D), jnp.float32),
)


def kernel(feat, edge_index, W1, b1, W2, b2):
    ei = edge_index.astype(jnp.int32).reshape(2 * E)
    deg_out, deg_in = _deg_kernel(ei)
    do = deg_out.reshape(N, 1)
    di = deg_in.reshape(N, 1)
    h0, h1 = _mm1(feat, do, W1)
    a0, a1 = _agg_kernel(h0, h1, ei)
    h0, h1 = _mid(a0, a1, di, do, b1.reshape(1, D), W2)
    a0, a1 = _agg_kernel(h0, h1, ei)
    return _fin(a0, a1, di, b2.reshape(1, D))


# speedup vs baseline: 10.3913x; 1.0010x over previous
import functools

import jax
import jax.numpy as jnp
from jax import lax
from jax.experimental import pallas as pl
from jax.experimental.pallas import tpu as pltpu
from jax.experimental.pallas import tpu_sc as plsc

N = 10000
E = 160000
D = 256
DH = 128

NC = 2
NS = 16
EPT = E // NS
GSZ = 40
NG = EPT // GSZ
NBUF = 9

_MESH = plsc.VectorSubcoreMesh(core_axis_name="c", subcore_axis_name="s")


_ZROWS = 2000


@functools.partial(
    pl.kernel,
    out_type=(jax.ShapeDtypeStruct((N,), jnp.float32),
              jax.ShapeDtypeStruct((N,), jnp.float32)),
    mesh=_MESH,
    scratch_types=[
        pltpu.VMEM_SHARED((N,), jnp.float32),
        pltpu.VMEM((EPT,), jnp.int32),
        pltpu.VMEM((EPT,), jnp.float32),
        pltpu.VMEM((_ZROWS,), jnp.float32),
        pltpu.SemaphoreType.DMA,
    ],
)
def _deg_kernel(ei_hbm, do_hbm, di_hbm, acc, idx_v, ones_v, zline_v, isem):
    c = lax.axis_index("c")
    s = lax.axis_index("s")

    ipre = pltpu.make_async_copy(
        ei_hbm.at[pl.ds(c * E + s * EPT, EPT)], idx_v, isem)
    ipre.start()

    @pl.loop(0, EPT // 16)
    def _(i):
        ones_v[pl.ds(i * 16, 16)] = jnp.ones((16,), jnp.float32)

    @pl.when(s < N // _ZROWS)
    def _():
        @pl.loop(0, _ZROWS // 16)
        def _(i):
            zline_v[pl.ds(i * 16, 16)] = jnp.zeros((16,), jnp.float32)
        pltpu.sync_copy(zline_v, acc.at[pl.ds(s * _ZROWS, _ZROWS)])

    ipre.wait()
    plsc.subcore_barrier()

    pltpu.sync_copy(ones_v, acc.at[idx_v], add=True)

    plsc.subcore_barrier()

    @pl.when(s < N // _ZROWS)
    def _():
        pltpu.sync_copy(acc.at[pl.ds(s * _ZROWS, _ZROWS)], zline_v)

        @pl.when(c == 0)
        def _():
            pltpu.sync_copy(zline_v, do_hbm.at[pl.ds(s * _ZROWS, _ZROWS)])

        @pl.when(c == 1)
        def _():
            pltpu.sync_copy(zline_v, di_hbm.at[pl.ds(s * _ZROWS, _ZROWS)])


_ORPB = GSZ
_ONB = N // _ORPB


def _sems(n):
    return [pltpu.SemaphoreType.DMA] * n


@functools.partial(
    pl.kernel,
    out_type=(jax.ShapeDtypeStruct((N, DH), jnp.float32),
              jax.ShapeDtypeStruct((N, DH), jnp.float32)),
    mesh=_MESH,
    scratch_types=[
        pltpu.VMEM_SHARED((N, DH), jnp.float32),
        pltpu.VMEM((NBUF, GSZ), jnp.int32),
        pltpu.VMEM((NBUF, GSZ), jnp.int32),
        pltpu.VMEM((NBUF, GSZ, DH), jnp.float32),
    ] + _sems(3 * NBUF + 3),
)
def _agg_kernel(h0_hbm, h1_hbm, ei_hbm, out0_hbm, out1_hbm,
                acc, sidx_v, didx_v, gbuf_v, *sems):
    c = lax.axis_index("c")
    s = lax.axis_index("s")
    isems = sems[0:NBUF]
    gsems = sems[NBUF:2 * NBUF]
    ssems = sems[2 * NBUF:3 * NBUF]
    zsem = sems[3 * NBUF]
    osems = sems[3 * NBUF + 1:3 * NBUF + 3]

    zb0 = gbuf_v.at[0]

    @pl.loop(0, GSZ * (DH // 16))
    def _(i):
        zb0[i // (DH // 16), pl.ds((i % (DH // 16)) * 16, 16)] = \
            jnp.zeros((16,), jnp.float32)

    @pl.loop(0, -(-_ONB // NS))
    def _(j):
        blk = s + j * NS

        @pl.when(blk < _ONB)
        def _():
            pltpu.make_async_copy(
                zb0, acc.at[pl.ds(blk * _ORPB, _ORPB)], zsem).start()

    @pl.loop(0, -(-_ONB // NS))
    def _(j):
        blk = s + j * NS

        @pl.when(blk < _ONB)
        def _():
            pltpu.make_async_copy(
                zb0, acc.at[pl.ds(blk * _ORPB, _ORPB)], zsem).wait()

    plsc.subcore_barrier()

    def _idx_descs(g, b):
        base = s * EPT + g * GSZ
        return (pltpu.make_async_copy(ei_hbm.at[pl.ds(base, GSZ)],
                                      sidx_v.at[b], isems[b]),
                pltpu.make_async_copy(ei_hbm.at[pl.ds(E + base, GSZ)],
                                      didx_v.at[b], isems[b]))

    def _idx_start(g, b):
        d0, d1 = _idx_descs(g, b)
        d0.start()
        d1.start()

    def _idx_wait(g, b):
        d0, d1 = _idx_descs(g, b)
        d0.wait()
        d1.wait()

    def _gather_start(g, b):
        @pl.when(c == 0)
        def _():
            pltpu.make_async_copy(
                h0_hbm.at[sidx_v.at[b]], gbuf_v.at[b], gsems[b]).start()

        @pl.when(c == 1)
        def _():
            pltpu.make_async_copy(
                h1_hbm.at[sidx_v.at[b]], gbuf_v.at[b], gsems[b]).start()

    def _gather_wait(g, b):
        @pl.when(c == 0)
        def _():
            pltpu.make_async_copy(
                h0_hbm.at[sidx_v.at[b]], gbuf_v.at[b], gsems[b]).wait()

        @pl.when(c == 1)
        def _():
            pltpu.make_async_copy(
                h1_hbm.at[sidx_v.at[b]], gbuf_v.at[b], gsems[b]).wait()

    def _scatter_desc(g, b):
        return pltpu.make_async_copy(
            gbuf_v.at[b], acc.at[didx_v.at[b]], ssems[b])

    for k in range(NBUF - 1):
        _idx_start(k, k)
    for k in range(NBUF - 2):
        _idx_wait(k, k)
        _gather_start(k, k)

    @pl.loop(0, -(-NG // NBUF))
    def _(t):
        for b in range(NBUF):
            g = t * NBUF + b

            @pl.when(g < NG)
            def _():
                _gather_wait(g, b)
                _scatter_desc(g, b).start(add=True)

                j1 = g + NBUF - 1
                b1 = (b + NBUF - 1) % NBUF

                @pl.when(j1 < NG)
                def _():
                    @pl.when(g >= 1)
                    def _():
                        _scatter_desc(g - 1, b1).wait()

                    _idx_start(j1, b1)

                j2 = g + NBUF - 2
                b2 = (b + NBUF - 2) % NBUF

                @pl.when(j2 < NG)
                def _():
                    _idx_wait(j2, b2)
                    _gather_start(j2, b2)

    for k in range(NG - NBUF, NG):
        _scatter_desc(k, k % NBUF).wait()

    plsc.subcore_barrier()

    def _out_desc(blk, b):
        base = blk * _ORPB

        @pl.when(c == 0)
        def _():
            pltpu.make_async_copy(
                gbuf_v.at[b], out0_hbm.at[pl.ds(base, _ORPB)],
                osems[b]).start()

        @pl.when(c == 1)
        def _():
            pltpu.make_async_copy(
                gbuf_v.at[b], out1_hbm.at[pl.ds(base, _ORPB)],
                osems[b]).start()

    def _out_wait(blk, b):
        base = blk * _ORPB

        @pl.when(c == 0)
        def _():
            pltpu.make_async_copy(
                gbuf_v.at[b], out0_hbm.at[pl.ds(base, _ORPB)],
                osems[b]).wait()

        @pl.when(c == 1)
        def _():
            pltpu.make_async_copy(
                gbuf_v.at[b], out1_hbm.at[pl.ds(base, _ORPB)],
                osems[b]).wait()

    @pl.loop(0, -(-_ONB // NS) // 2 + 1)
    def _(t):
        for b in range(2):
            j = t * 2 + b
            blk = s + j * NS

            @pl.when(blk < _ONB)
            def _():
                @pl.when(j >= 2)
                def _():
                    _out_wait(blk - 2 * NS, b)

                pltpu.sync_copy(acc.at[pl.ds(blk * _ORPB, _ORPB)],
                                gbuf_v.at[b])
                _out_desc(blk, b)

    for b in range(2):
        last_j = lax.div(_ONB - 1 - s, NS)
        blk = s + last_j * NS
        blk_b = jnp.where(lax.rem(last_j, 2) == b, blk, blk - NS)

        @pl.when(blk_b >= 0)
        def _():
            _out_wait(blk_b, b)


_BLK = 1000
_GRID = N // _BLK


def _norm(deg):
    return jnp.where(deg > 0, lax.rsqrt(jnp.maximum(deg, 1e-12)), 0.0)


def _mm1_body(x_ref, do_ref, w_ref, o0_ref, o1_ref):
    ns = _norm(do_ref[...])
    h = jnp.dot((x_ref[...] * ns).astype(jnp.bfloat16),
                w_ref[...].astype(jnp.bfloat16),
                preferred_element_type=jnp.float32)
    o0_ref[...] = h[:, :DH]
    o1_ref[...] = h[:, DH:]


_mm1 = pl.pallas_call(
    _mm1_body,
    grid=(_GRID,),
    in_specs=[
        pl.BlockSpec((_BLK, D), lambda i: (i, 0)),
        pl.BlockSpec((_BLK, 1), lambda i: (i, 0)),
        pl.BlockSpec((D, D), lambda i: (0, 0)),
    ],
    out_specs=[pl.BlockSpec((_BLK, DH), lambda i: (i, 0)),
               pl.BlockSpec((_BLK, DH), lambda i: (i, 0))],
    out_shape=(jax.ShapeDtypeStruct((N, DH), jnp.float32),
               jax.ShapeDtypeStruct((N, DH), jnp.float32)),
)


def _mid_body(a0_ref, a1_ref, di_ref, do_ref, b_ref, w_ref, o0_ref, o1_ref):
    nd = _norm(di_ref[...])
    ns = _norm(do_ref[...])
    t0 = jnp.maximum(a0_ref[...] * nd + b_ref[0, :DH], 0.0) * ns
    t1 = jnp.maximum(a1_ref[...] * nd + b_ref[0, DH:], 0.0) * ns
    wb = w_ref[...].astype(jnp.bfloat16)
    h = (jnp.dot(t0.astype(jnp.bfloat16), wb[:DH, :],
                 preferred_element_type=jnp.float32)
         + jnp.dot(t1.astype(jnp.bfloat16), wb[DH:, :],
                   preferred_element_type=jnp.float32))
    o0_ref[...] = h[:, :DH]
    o1_ref[...] = h[:, DH:]


_mid = pl.pallas_call(
    _mid_body,
    grid=(_GRID,),
    in_specs=[
        pl.BlockSpec((_BLK, DH), lambda i: (i, 0)),
        pl.BlockSpec((_BLK, DH), lambda i: (i, 0)),
        pl.BlockSpec((_BLK, 1), lambda i: (i, 0)),
        pl.BlockSpec((_BLK, 1), lambda i: (i, 0)),
        pl.BlockSpec((1, D), lambda i: (0, 0)),
        pl.BlockSpec((D, D), lambda i: (0, 0)),
    ],
    out_specs=[pl.BlockSpec((_BLK, DH), lambda i: (i, 0)),
               pl.BlockSpec((_BLK, DH), lambda i: (i, 0))],
    out_shape=(jax.ShapeDtypeStruct((N, DH), jnp.float32),
               jax.ShapeDtypeStruct((N, DH), jnp.float32)),
)


def _fin_body(a0_ref, a1_ref, di_ref, b_ref, o_ref):
    nd = _norm(di_ref[...])
    t0 = jnp.maximum(a0_ref[...] * nd + b_ref[0, :DH], 0.0)
    t1 = jnp.maximum(a1_ref[...] * nd + b_ref[0, DH:], 0.0)
    o_ref[...] = jnp.concatenate([t0, t1], axis=1)


_fin = pl.pallas_call(
    _fin_body,
    grid=(_GRID,),
    in_specs=[
        pl.BlockSpec((_BLK, DH), lambda i: (i, 0)),
        pl.BlockSpec((_BLK, DH), lambda i: (i, 0)),
        pl.BlockSpec((_BLK, 1), lambda i: (i, 0)),
        pl.BlockSpec((1, D), lambda i: (0, 0)),
    ],
    out_specs=pl.BlockSpec((_BLK, D), lambda i: (i, 0)),
    out_shape=jax.ShapeDtypeStruct((N, D), jnp.float32),
)


def kernel(feat, edge_index, W1, b1, W2, b2):
    ei = edge_index.astype(jnp.int32).reshape(2 * E)
    deg_out, deg_in = _deg_kernel(ei)
    do = deg_out.reshape(N, 1)
    di = deg_in.reshape(N, 1)
    h0, h1 = _mm1(feat, do, W1)
    a0, a1 = _agg_kernel(h0, h1, ei)
    h0, h1 = _mid(a0, a1, di, do, b1.reshape(1, D), W2)
    a0, a1 = _agg_kernel(h0, h1, ei)
    return _fin(a0, a1, di, b2.reshape(1, D))
